# baseline (speedup 1.0000x reference)
import jax
import jax.numpy as jnp
from jax import lax
from jax.experimental import pallas as pl
from jax.experimental.pallas import tpu as pltpu
from jax.experimental.pallas import tpu_sc as plsc

f32 = jnp.float32
i32 = jnp.int32

NC = 2
NS = 16
NT = NC * NS
LANES = 16
EB = 400
NB = 4


_MESH = plsc.VectorSubcoreMesh(core_axis_name="c", subcore_axis_name="s")

_CP = pltpu.CompilerParams(needs_layout_passes=False)


def _edge_pass(n_pad, e, first, h_two_rows=True):
    C = n_pad // NS
    EPT = e // NT

    out_type = jax.ShapeDtypeStruct((2 * n_pad,), f32) if first else (
        jax.ShapeDtypeStruct((2 * n_pad,), f32),
        jax.ShapeDtypeStruct((2 * n_pad,), f32),
    )

    scratch = [
        pltpu.VMEM_SHARED((n_pad,), f32),
        pltpu.VMEM((n_pad,), f32),
        pltpu.VMEM((C,), f32),
        pltpu.VMEM((C,), f32),
        [pltpu.VMEM((EB,), i32)] * NB,
        [pltpu.VMEM((EB,), i32)] * NB,
        [pltpu.VMEM((EB,), f32)] * NB,
        [pltpu.SemaphoreType.DMA] * NB,
        [pltpu.SemaphoreType.DMA] * NB,
        pltpu.VMEM((LANES,), f32),
    ]

    def body(*refs):
        if first:
            (h_ref, src_ref, dst_ref, pout_ref,
             agg_sh, h_tile, hbuf, q0,
             sbufs, dbufs, vbufs, sl, ss, wbuf) = refs
        else:
            (h_ref, p_ref, w_ref, src_ref, dst_ref, hrep_ref, pout_ref,
             agg_sh, h_tile, hbuf, q0,
             sbufs, dbufs, vbufs, sl, ss, wbuf) = refs
        c = lax.axis_index("c")
        s = lax.axis_index("s")
        base = s * C

        if not first:
            hoff = c * n_pad + base if h_two_rows else base
            pltpu.sync_copy(h_ref.at[pl.ds(hoff, C)], hbuf)
            pltpu.sync_copy(w_ref, wbuf)
            pltpu.sync_copy(p_ref.at[pl.ds(base, C)], q0)
            wv = wbuf[...]

            @pl.loop(0, C, step=LANES)
            def _(i):
                sl_ = pl.ds(i, LANES)
                hbuf[sl_] = hbuf[sl_] + q0[sl_]

            pltpu.sync_copy(p_ref.at[pl.ds(n_pad + base, C)], q0)

            @pl.loop(0, C, step=LANES)
            def _(i):
                sl_ = pl.ds(i, LANES)
                hbuf[sl_] = (hbuf[sl_] + q0[sl_]) * wv
                q0[sl_] = jnp.zeros((LANES,), f32)

            pltpu.sync_copy(hbuf, hrep_ref.at[pl.ds(c * n_pad + base, C)])
        else:
            @pl.loop(0, C, step=LANES)
            def _(i):
                q0[pl.ds(i, LANES)] = jnp.zeros((LANES,), f32)

        pltpu.sync_copy(q0, agg_sh.at[pl.ds(base, C)])

        plsc.subcore_barrier()

        if first:
            pltpu.sync_copy(h_ref, h_tile)
        else:
            pltpu.sync_copy(hrep_ref.at[pl.ds(c * n_pad, n_pad)], h_tile)

        ebase = (c * NS + s) * EPT
        NBLK = EPT // EB
        assert NBLK >= 4 and (NBLK - 4) % 4 == 0

        def start_loads(blk, b):
            off = ebase + blk * EB
            pltpu.async_copy(src_ref.at[pl.ds(off, EB)], sbufs[b], sl[b])
            pltpu.async_copy(dst_ref.at[pl.ds(off, EB)], dbufs[b], sl[b])

        def wait_loads(b):
            pltpu.make_async_copy(
                src_ref.at[pl.ds(0, EB)], sbufs[b], sl[b]).wait()
            pltpu.make_async_copy(
                dst_ref.at[pl.ds(0, EB)], dbufs[b], sl[b]).wait()

        def gather(b):
            @pl.loop(0, EB, step=LANES)
            def _(i):
                idx = sbufs[b][pl.ds(i, LANES)]
                vbufs[b][pl.ds(i, LANES)] = plsc.load_gather(h_tile, [idx])

        def start_scatter(b):
            pltpu.async_copy(vbufs[b], agg_sh.at[dbufs[b]], ss[b], add=True)

        def wait_scatter(b):
            pltpu.make_async_copy(vbufs[b], agg_sh.at[dbufs[b]], ss[b]).wait()

        start_loads(0, 0)
        start_loads(1, 1)
        for blk in (0, 1):
            wait_loads(blk)
            gather(blk)
            start_scatter(blk)
            start_loads(blk + 2, blk + 2)

        @pl.loop(2, NBLK - 2, step=4)
        def _(g):
            for j in range(4):
                b = (2 + j) % 4
                wait_loads(b)
                gather(b)
                start_scatter(b)
                wait_scatter((b + 2) % 4)
                start_loads(g + j + 2, (b + 2) % 4)

        for blk in (NBLK - 2, NBLK - 1):
            b = blk % 4
            wait_loads(b)
            gather(b)
            start_scatter(b)
            wait_scatter((b + 2) % 4)
        wait_scatter((NBLK - 2) % 4)
        wait_scatter((NBLK - 1) % 4)

        plsc.subcore_barrier()

        pltpu.sync_copy(agg_sh.at[pl.ds(base, C)], q0)
        pltpu.sync_copy(q0, pout_ref.at[pl.ds(c * n_pad + base, C)])

    return pl.kernel(body, out_type=out_type, mesh=_MESH,
                     scratch_types=scratch, compiler_params=_CP)


def _final_combine(n_pad):
    C = n_pad // NS

    scratch = [
        pltpu.VMEM((C,), f32),
        pltpu.VMEM((C,), f32),
        pltpu.VMEM((C,), f32),
        pltpu.VMEM((LANES,), f32),
    ]

    def body(h_ref, p_ref, w_ref, hout_ref, hbuf, q0, q1, wbuf):
        c = lax.axis_index("c")
        s = lax.axis_index("s")
        base = s * C

        @pl.when(c == 0)
        def _():
            pltpu.sync_copy(w_ref, wbuf)
            pltpu.sync_copy(h_ref.at[pl.ds(base, C)], hbuf)
            pltpu.sync_copy(p_ref.at[pl.ds(base, C)], q0)
            pltpu.sync_copy(p_ref.at[pl.ds(n_pad + base, C)], q1)
            wv = wbuf[...]

            @pl.loop(0, C, step=LANES)
            def _(i):
                sl = pl.ds(i, LANES)
                hbuf[sl] = (hbuf[sl] + q0[sl] + q1[sl]) * wv

            pltpu.sync_copy(hbuf, hout_ref.at[pl.ds(base, C)])

    return pl.kernel(body, out_type=jax.ShapeDtypeStruct((n_pad,), f32),
                     mesh=_MESH, scratch_types=scratch, compiler_params=_CP)


def kernel(x, edge_index, W):
    n = x.shape[0]
    e = edge_index.shape[1]
    num_hop = W.shape[0]
    n_pad = -(-n // (NS * LANES)) * (NS * LANES)
    assert e % (NT * EB) == 0

    src = edge_index[0].astype(i32)
    dst = edge_index[1].astype(i32)
    wv = [jnp.broadcast_to(W[k, 0, 0].astype(f32), (LANES,))
          for k in range(num_hop)]

    h = jnp.zeros((n_pad,), f32).at[:n].set(x[:, 0])

    p = _edge_pass(n_pad, e, True)(h, src, dst)
    hrep = None
    for k in range(1, num_hop):
        if hrep is None:
            hrep, p = _edge_pass(n_pad, e, False, h_two_rows=False)(
                h, p, wv[k - 1], src, dst)
        else:
            hrep, p = _edge_pass(n_pad, e, False, h_two_rows=True)(
                hrep, p, wv[k - 1], src, dst)
    h_last = h if hrep is None else hrep[:n_pad]
    h_out = _final_combine(n_pad)(h_last, p, wv[num_hop - 1])

    return h_out[:n].reshape(n, 1)


# speedup vs baseline: 1.3785x; 1.3785x over previous
import jax
import jax.numpy as jnp
from jax import lax
from jax.experimental import pallas as pl
from jax.experimental.pallas import tpu as pltpu
from jax.experimental.pallas import tpu_sc as plsc

f32 = jnp.float32
i32 = jnp.int32

NC = 2
NS = 16
NT = NC * NS
LANES = 16
EB = 4000
NB = 4


_MESH = plsc.VectorSubcoreMesh(core_axis_name="c", subcore_axis_name="s")

_CP = pltpu.CompilerParams(needs_layout_passes=False)


def _edge_pass(n_pad, e, first, h_two_rows=True):
    C = n_pad // NS
    EPT = e // NT

    out_type = jax.ShapeDtypeStruct((2 * n_pad,), f32) if first else (
        jax.ShapeDtypeStruct((2 * n_pad,), f32),
        jax.ShapeDtypeStruct((2 * n_pad,), f32),
    )

    scratch = [
        pltpu.VMEM_SHARED((n_pad,), f32),
        pltpu.VMEM_SHARED((n_pad,), f32),
        pltpu.VMEM((C,), f32),
        pltpu.VMEM((C,), f32),
        [pltpu.VMEM((EB,), i32)] * NB,
        [pltpu.VMEM((EB,), i32)] * NB,
        [pltpu.VMEM((EB,), f32)] * NB,
        [pltpu.SemaphoreType.DMA] * NB,
        [pltpu.SemaphoreType.DMA] * NB,
        pltpu.VMEM((LANES,), f32),
    ]

    def body(*refs):
        if first:
            (h_ref, src_ref, dst_ref, pout_ref,
             agg_sh, h_sh, hbuf, q0,
             sbufs, dbufs, vbufs, sl, ss, wbuf) = refs
        else:
            (h_ref, p_ref, w_ref, src_ref, dst_ref, hrep_ref, pout_ref,
             agg_sh, h_sh, hbuf, q0,
             sbufs, dbufs, vbufs, sl, ss, wbuf) = refs
        c = lax.axis_index("c")
        s = lax.axis_index("s")
        base = s * C

        if not first:
            hoff = c * n_pad + base if h_two_rows else base
            pltpu.sync_copy(h_ref.at[pl.ds(hoff, C)], hbuf)
            pltpu.sync_copy(w_ref, wbuf)
            pltpu.sync_copy(p_ref.at[pl.ds(base, C)], q0)
            wv = wbuf[...]

            @pl.loop(0, C, step=LANES)
            def _(i):
                sl_ = pl.ds(i, LANES)
                hbuf[sl_] = hbuf[sl_] + q0[sl_]

            pltpu.sync_copy(p_ref.at[pl.ds(n_pad + base, C)], q0)

            @pl.loop(0, C, step=LANES)
            def _(i):
                sl_ = pl.ds(i, LANES)
                hbuf[sl_] = (hbuf[sl_] + q0[sl_]) * wv
                q0[sl_] = jnp.zeros((LANES,), f32)

            pltpu.sync_copy(hbuf, hrep_ref.at[pl.ds(c * n_pad + base, C)])
        else:
            pltpu.sync_copy(h_ref.at[pl.ds(base, C)], hbuf)

            @pl.loop(0, C, step=LANES)
            def _(i):
                q0[pl.ds(i, LANES)] = jnp.zeros((LANES,), f32)

        pltpu.sync_copy(hbuf, h_sh.at[pl.ds(base, C)])
        pltpu.sync_copy(q0, agg_sh.at[pl.ds(base, C)])

        plsc.subcore_barrier()

        ebase = (c * NS + s) * EPT
        NBLK = EPT // EB
        MAIN = (NBLK - 4) // 4 * 4
        assert NBLK >= 6

        def start_loads(blk, b):
            off = ebase + blk * EB
            pltpu.async_copy(src_ref.at[pl.ds(off, EB)], sbufs[b], sl[b])
            pltpu.async_copy(dst_ref.at[pl.ds(off, EB)], dbufs[b], sl[b])

        def wait_loads(b):
            pltpu.make_async_copy(
                src_ref.at[pl.ds(0, EB)], sbufs[b], sl[b]).wait()
            pltpu.make_async_copy(
                dst_ref.at[pl.ds(0, EB)], dbufs[b], sl[b]).wait()

        def gather(b):
            pltpu.sync_copy(h_sh.at[sbufs[b]], vbufs[b])

        def start_scatter(b):
            pltpu.async_copy(vbufs[b], agg_sh.at[dbufs[b]], ss[b], add=True)

        def wait_scatter(b):
            pltpu.make_async_copy(vbufs[b], agg_sh.at[dbufs[b]], ss[b]).wait()

        start_loads(0, 0)
        start_loads(1, 1)
        for blk in (0, 1):
            wait_loads(blk)
            gather(blk)
            start_scatter(blk)
            start_loads(blk + 2, blk + 2)

        @pl.loop(2, 2 + MAIN, step=4)
        def _(g):
            for j in range(4):
                b = (2 + j) % 4
                wait_loads(b)
                gather(b)
                start_scatter(b)
                wait_scatter((b + 2) % 4)
                start_loads(g + j + 2, (b + 2) % 4)

        for blk in range(2 + MAIN, NBLK):
            b = blk % 4
            wait_loads(b)
            gather(b)
            start_scatter(b)
            wait_scatter((b + 2) % 4)
            if blk + 2 < NBLK:
                start_loads(blk + 2, (blk + 2) % 4)
        wait_scatter((NBLK - 2) % 4)
        wait_scatter((NBLK - 1) % 4)

        plsc.subcore_barrier()

        pltpu.sync_copy(agg_sh.at[pl.ds(base, C)], q0)
        pltpu.sync_copy(q0, pout_ref.at[pl.ds(c * n_pad + base, C)])

    return pl.kernel(body, out_type=out_type, mesh=_MESH,
                     scratch_types=scratch, compiler_params=_CP)


def _final_combine(n_pad):
    C = n_pad // NS

    scratch = [
        pltpu.VMEM((C,), f32),
        pltpu.VMEM((C,), f32),
        pltpu.VMEM((C,), f32),
        pltpu.VMEM((LANES,), f32),
    ]

    def body(h_ref, p_ref, w_ref, hout_ref, hbuf, q0, q1, wbuf):
        c = lax.axis_index("c")
        s = lax.axis_index("s")
        base = s * C

        @pl.when(c == 0)
        def _():
            pltpu.sync_copy(w_ref, wbuf)
            pltpu.sync_copy(h_ref.at[pl.ds(base, C)], hbuf)
            pltpu.sync_copy(p_ref.at[pl.ds(base, C)], q0)
            pltpu.sync_copy(p_ref.at[pl.ds(n_pad + base, C)], q1)
            wv = wbuf[...]

            @pl.loop(0, C, step=LANES)
            def _(i):
                sl = pl.ds(i, LANES)
                hbuf[sl] = (hbuf[sl] + q0[sl] + q1[sl]) * wv

            pltpu.sync_copy(hbuf, hout_ref.at[pl.ds(base, C)])

    return pl.kernel(body, out_type=jax.ShapeDtypeStruct((n_pad,), f32),
                     mesh=_MESH, scratch_types=scratch, compiler_params=_CP)


def kernel(x, edge_index, W):
    n = x.shape[0]
    e = edge_index.shape[1]
    num_hop = W.shape[0]
    n_pad = -(-n // (NS * LANES)) * (NS * LANES)
    assert e % (NT * EB) == 0

    src = edge_index[0].astype(i32)
    dst = edge_index[1].astype(i32)
    wv = [jnp.broadcast_to(W[k, 0, 0].astype(f32), (LANES,))
          for k in range(num_hop)]

    h = jnp.zeros((n_pad,), f32).at[:n].set(x[:, 0])

    p = _edge_pass(n_pad, e, True)(h, src, dst)
    hrep = None
    for k in range(1, num_hop):
        if hrep is None:
            hrep, p = _edge_pass(n_pad, e, False, h_two_rows=False)(
                h, p, wv[k - 1], src, dst)
        else:
            hrep, p = _edge_pass(n_pad, e, False, h_two_rows=True)(
                hrep, p, wv[k - 1], src, dst)
    h_last = h if hrep is None else hrep[:n_pad]
    h_out = _final_combine(n_pad)(h_last, p, wv[num_hop - 1])

    return h_out[:n].reshape(n, 1)


# speedup vs baseline: 1.4018x; 1.0169x over previous
import jax
import jax.numpy as jnp
from jax import lax
from jax.experimental import pallas as pl
from jax.experimental.pallas import tpu as pltpu
from jax.experimental.pallas import tpu_sc as plsc

f32 = jnp.float32
i32 = jnp.int32

NC = 2
NS = 16
NT = NC * NS
LANES = 16
EB = 4000
NB = 4
FW = 16

_MESH = plsc.VectorSubcoreMesh(core_axis_name="c", subcore_axis_name="s")

_CP = pltpu.CompilerParams(needs_layout_passes=False)


def _ring_scratch():
    return [
        [pltpu.VMEM((EB,), i32)] * NB,
        [pltpu.VMEM((EB,), i32)] * NB,
        [pltpu.VMEM((EB,), f32)] * NB,
        [pltpu.SemaphoreType.DMA] * NB,
        [pltpu.SemaphoreType.DMA] * NB,
    ]


def _edge_ring(src_ref, dst_ref, h_sh, agg_sh, sbufs, dbufs, vbufs, sl, ss,
               ebase, ept):
    NBLK = ept // EB
    MAIN = (NBLK - 4) // 4 * 4
    assert NBLK >= 6

    def start_loads(blk, b):
        off = ebase + blk * EB
        pltpu.async_copy(src_ref.at[pl.ds(off, EB)], sbufs[b], sl[b])
        pltpu.async_copy(dst_ref.at[pl.ds(off, EB)], dbufs[b], sl[b])

    def wait_loads(b):
        pltpu.make_async_copy(src_ref.at[pl.ds(0, EB)], sbufs[b], sl[b]).wait()
        pltpu.make_async_copy(dst_ref.at[pl.ds(0, EB)], dbufs[b], sl[b]).wait()

    def gather(b):
        pltpu.sync_copy(h_sh.at[sbufs[b]], vbufs[b])

    def start_scatter(b):
        pltpu.async_copy(vbufs[b], agg_sh.at[dbufs[b]], ss[b], add=True)

    def wait_scatter(b):
        pltpu.make_async_copy(vbufs[b], agg_sh.at[dbufs[b]], ss[b]).wait()

    start_loads(0, 0)
    start_loads(1, 1)
    for blk in (0, 1):
        wait_loads(blk)
        gather(blk)
        start_scatter(blk)
        start_loads(blk + 2, blk + 2)

    @pl.loop(2, 2 + MAIN, step=4)
    def _(g):
        for j in range(4):
            b = (2 + j) % 4
            wait_loads(b)
            gather(b)
            start_scatter(b)
            wait_scatter((b + 2) % 4)
            start_loads(g + j + 2, (b + 2) % 4)

    for blk in range(2 + MAIN, NBLK):
        b = blk % 4
        wait_loads(b)
        gather(b)
        start_scatter(b)
        wait_scatter((b + 2) % 4)
        if blk + 2 < NBLK:
            start_loads(blk + 2, (blk + 2) % 4)
    wait_scatter((NBLK - 2) % 4)
    wait_scatter((NBLK - 1) % 4)


def _hop0(n_pad, e, n_flag_slots):
    C = n_pad // NS
    EPT = e // NT

    out_type = (
        jax.ShapeDtypeStruct((2 * n_pad,), f32),
        jax.ShapeDtypeStruct((2 * n_flag_slots * FW,), i32),
    )
    scratch = [
        pltpu.VMEM_SHARED((n_pad,), f32),
        pltpu.VMEM_SHARED((n_pad,), f32),
        pltpu.VMEM((C,), f32),
        pltpu.VMEM((C,), f32),
        pltpu.VMEM((FW,), i32),
    ] + _ring_scratch()

    def body(h_ref, src_ref, dst_ref, pout_ref, flags_ref,
             agg_sh, h_sh, hbuf, q0, fbuf, sbufs, dbufs, vbufs, sl, ss):
        c = lax.axis_index("c")
        s = lax.axis_index("s")
        base = s * C

        pltpu.sync_copy(h_ref.at[pl.ds(base, C)], hbuf)

        @pl.loop(0, C, step=LANES)
        def _(i):
            q0[pl.ds(i, LANES)] = jnp.zeros((LANES,), f32)

        pltpu.sync_copy(hbuf, h_sh.at[pl.ds(base, C)])
        pltpu.sync_copy(q0, agg_sh.at[pl.ds(base, C)])

        @pl.when(s == 0)
        def _():
            fbuf[...] = jnp.zeros((FW,), i32)
            for k in range(n_flag_slots):
                pltpu.sync_copy(
                    fbuf,
                    flags_ref.at[pl.ds((c * n_flag_slots + k) * FW, FW)])

        plsc.subcore_barrier()
        _edge_ring(src_ref, dst_ref, h_sh, agg_sh,
                   sbufs, dbufs, vbufs, sl, ss, (c * NS + s) * EPT, EPT)
        plsc.subcore_barrier()

        pltpu.sync_copy(agg_sh.at[pl.ds(base, C)], q0)
        pltpu.sync_copy(q0, pout_ref.at[pl.ds(c * n_pad + base, C)])

    return pl.kernel(body, out_type=out_type, mesh=_MESH,
                     scratch_types=scratch, compiler_params=_CP)


def _fused(n_pad, e, num_hop):
    C = n_pad // NS
    EPT = e // NT
    n_flag_slots = num_hop - 1

    out_type = tuple(
        [jax.ShapeDtypeStruct((n_pad,), f32)]
        + [jax.ShapeDtypeStruct((2 * n_pad,), f32)] * n_flag_slots)
    scratch = [
        pltpu.VMEM_SHARED((n_pad,), f32),
        pltpu.VMEM_SHARED((n_pad,), f32),
        pltpu.VMEM((C,), f32),
        pltpu.VMEM((C,), f32),
        pltpu.VMEM((LANES,), f32),
        pltpu.VMEM((FW,), i32),
        pltpu.SemaphoreType.DMA,
    ] + _ring_scratch()

    def body(*refs):
        (h_ref, p0_ref, w_ref, src_ref, dst_ref, flags_ref,
         out_ref, *rest) = refs
        pouts_out = rest[:n_flag_slots]
        (agg_sh, h_sh, hbuf, q0, wbuf, fbuf, sf,
         sbufs, dbufs, vbufs, sl, ss) = rest[n_flag_slots:]
        c = lax.axis_index("c")
        s = lax.axis_index("s")
        base = s * C
        psrcs = [p0_ref] + list(pouts_out)

        def combine(psrc, k):
            pltpu.sync_copy(w_ref.at[pl.ds(k * LANES, LANES)], wbuf)
            pltpu.sync_copy(psrc.at[pl.ds(base, C)], q0)

            @pl.loop(0, C, step=LANES)
            def _(i):
                sl_ = pl.ds(i, LANES)
                hbuf[sl_] = hbuf[sl_] + q0[sl_]

            pltpu.sync_copy(psrc.at[pl.ds(n_pad + base, C)], q0)
            wv = wbuf[...]

            @pl.loop(0, C, step=LANES)
            def _(i):
                sl_ = pl.ds(i, LANES)
                hbuf[sl_] = (hbuf[sl_] + q0[sl_]) * wv
                q0[sl_] = jnp.zeros((LANES,), f32)

        def flag_wait(k):
            off = ((1 - c) * n_flag_slots + (k - 1)) * FW

            def cond(carry):
                it, done = carry
                return jnp.logical_and(done == 0, it < jnp.int32(200000))

            def poll(carry):
                it, _ = carry
                pltpu.async_copy(flags_ref.at[pl.ds(off, FW)], fbuf, sf
                                 ).wait()
                got = jnp.max(jnp.where(fbuf[...] == k, 1, 0).astype(i32))
                return (it + jnp.int32(1), got)

            lax.while_loop(cond, poll, (jnp.int32(0), jnp.int32(0)))

        pltpu.sync_copy(h_ref.at[pl.ds(base, C)], hbuf)

        for k in range(1, num_hop):
            if k > 1:
                flag_wait(k - 1)
            combine(psrcs[k - 1], k - 1)
            pltpu.sync_copy(hbuf, h_sh.at[pl.ds(base, C)])
            pltpu.sync_copy(q0, agg_sh.at[pl.ds(base, C)])
            plsc.subcore_barrier()
            _edge_ring(src_ref, dst_ref, h_sh, agg_sh,
                       sbufs, dbufs, vbufs, sl, ss, (c * NS + s) * EPT, EPT)
            plsc.subcore_barrier()
            pltpu.sync_copy(agg_sh.at[pl.ds(base, C)], q0)
            pltpu.sync_copy(q0, psrcs[k].at[pl.ds(c * n_pad + base, C)])
            plsc.subcore_barrier()

            @pl.when(s == 0)
            def _():
                fbuf[...] = jnp.full((FW,), k, i32)
                pltpu.sync_copy(
                    fbuf,
                    flags_ref.at[pl.ds((c * n_flag_slots + k - 1) * FW, FW)])

        flag_wait(num_hop - 1)
        combine(psrcs[num_hop - 1], num_hop - 1)

        @pl.when(c == 0)
        def _():
            pltpu.sync_copy(hbuf, out_ref.at[pl.ds(base, C)])

    return pl.kernel(body, out_type=out_type, mesh=_MESH,
                     scratch_types=scratch, compiler_params=_CP)


def kernel(x, edge_index, W):
    n = x.shape[0]
    e = edge_index.shape[1]
    num_hop = W.shape[0]
    n_pad = -(-n // (NS * LANES)) * (NS * LANES)
    assert e % (NT * EB) == 0 and num_hop >= 2

    src = edge_index[0].astype(i32)
    dst = edge_index[1].astype(i32)
    wvec = jnp.broadcast_to(W[:, 0, 0].astype(f32)[:, None],
                            (num_hop, LANES)).reshape(-1)

    h = jnp.zeros((n_pad,), f32).at[:n].set(x[:, 0])

    p0, flags = _hop0(n_pad, e, num_hop - 1)(h, src, dst)
    out = _fused(n_pad, e, num_hop)(h, p0, wvec, src, dst, flags)[0]

    return out[:n].reshape(n, 1)


# speedup vs baseline: 1.4093x; 1.0053x over previous
import jax
import jax.numpy as jnp
from jax import lax
from jax.experimental import pallas as pl
from jax.experimental.pallas import tpu as pltpu
from jax.experimental.pallas import tpu_sc as plsc

f32 = jnp.float32
i32 = jnp.int32

NC = 2
NS = 16
NT = NC * NS
LANES = 16
EB = 4000
NB = 4
FW = 16

_MESH = plsc.VectorSubcoreMesh(core_axis_name="c", subcore_axis_name="s")

_CP = pltpu.CompilerParams(needs_layout_passes=False)


def _ring_scratch():
    return [
        [pltpu.VMEM((EB,), i32)] * NB,
        [pltpu.VMEM((EB,), i32)] * NB,
        [pltpu.VMEM((EB,), f32)] * NB,
        [pltpu.SemaphoreType.DMA] * NB,
        [pltpu.SemaphoreType.DMA] * NB,
    ]


def _edge_ring(src_ref, dst_ref, h_sh, agg_sh, sbufs, dbufs, vbufs, sl, ss,
               ebase, ept):
    NBLK = ept // EB
    MAIN = (NBLK - 4) // 4 * 4
    assert NBLK >= 6

    def start_loads(blk, b):
        off = ebase + blk * EB
        pltpu.async_copy(src_ref.at[pl.ds(off, EB)], sbufs[b], sl[b])
        pltpu.async_copy(dst_ref.at[pl.ds(off, EB)], dbufs[b], sl[b])

    def wait_loads(b):
        pltpu.make_async_copy(src_ref.at[pl.ds(0, EB)], sbufs[b], sl[b]).wait()
        pltpu.make_async_copy(dst_ref.at[pl.ds(0, EB)], dbufs[b], sl[b]).wait()

    def gather(b):
        pltpu.sync_copy(h_sh.at[sbufs[b]], vbufs[b])

    def start_scatter(b):
        pltpu.async_copy(vbufs[b], agg_sh.at[dbufs[b]], ss[b], add=True)

    def wait_scatter(b):
        pltpu.make_async_copy(vbufs[b], agg_sh.at[dbufs[b]], ss[b]).wait()

    start_loads(0, 0)
    start_loads(1, 1)
    for blk in (0, 1):
        wait_loads(blk)
        gather(blk)
        start_scatter(blk)
        start_loads(blk + 2, blk + 2)

    @pl.loop(2, 2 + MAIN, step=4)
    def _(g):
        for j in range(4):
            b = (2 + j) % 4
            wait_loads(b)
            gather(b)
            start_scatter(b)
            wait_scatter((b + 2) % 4)
            start_loads(g + j + 2, (b + 2) % 4)

    for blk in range(2 + MAIN, NBLK):
        b = blk % 4
        wait_loads(b)
        gather(b)
        start_scatter(b)
        wait_scatter((b + 2) % 4)
        if blk + 2 < NBLK:
            start_loads(blk + 2, (blk + 2) % 4)
    wait_scatter((NBLK - 2) % 4)
    wait_scatter((NBLK - 1) % 4)


def _all_hops(n_pad, e, num_hop):
    C = n_pad // NS
    EPT = e // NT
    n_flag_slots = num_hop

    out_type = tuple(
        [jax.ShapeDtypeStruct((n_pad,), f32)]
        + [jax.ShapeDtypeStruct((2 * n_pad,), f32)] * n_flag_slots)
    scratch = [
        pltpu.VMEM_SHARED((n_pad,), f32),
        pltpu.VMEM_SHARED((n_pad,), f32),
        pltpu.VMEM((C,), f32),
        pltpu.VMEM((C,), f32),
        pltpu.VMEM((LANES,), f32),
        pltpu.VMEM((FW,), i32),
        pltpu.SemaphoreType.DMA,
    ] + _ring_scratch()

    def body(*refs):
        (h_ref, w_ref, src_ref, dst_ref, flags_ref,
         out_ref, *rest) = refs
        pouts = rest[:n_flag_slots]
        (agg_sh, h_sh, hbuf, q0, wbuf, fbuf, sf,
         sbufs, dbufs, vbufs, sl, ss) = rest[n_flag_slots:]
        c = lax.axis_index("c")
        s = lax.axis_index("s")
        base = s * C

        def combine(psrc, k):
            pltpu.sync_copy(w_ref.at[pl.ds(k * LANES, LANES)], wbuf)
            pltpu.sync_copy(psrc.at[pl.ds(base, C)], q0)

            @pl.loop(0, C, step=LANES)
            def _(i):
                sl_ = pl.ds(i, LANES)
                hbuf[sl_] = hbuf[sl_] + q0[sl_]

            pltpu.sync_copy(psrc.at[pl.ds(n_pad + base, C)], q0)
            wv = wbuf[...]

            @pl.loop(0, C, step=LANES)
            def _(i):
                sl_ = pl.ds(i, LANES)
                hbuf[sl_] = (hbuf[sl_] + q0[sl_]) * wv
                q0[sl_] = jnp.zeros((LANES,), f32)

        def flag_wait(k):
            off = ((1 - c) * n_flag_slots + k) * FW

            def cond(carry):
                it, done = carry
                return jnp.logical_and(done == 0, it < jnp.int32(200000))

            def poll(carry):
                it, _ = carry
                pltpu.async_copy(flags_ref.at[pl.ds(off, FW)], fbuf, sf
                                 ).wait()
                got = jnp.max(
                    jnp.where(fbuf[...] == k + 1, 1, 0).astype(i32))
                return (it + jnp.int32(1), got)

            lax.while_loop(cond, poll, (jnp.int32(0), jnp.int32(0)))

        pltpu.sync_copy(h_ref.at[pl.ds(base, C)], hbuf)

        @pl.loop(0, C, step=LANES)
        def _(i):
            q0[pl.ds(i, LANES)] = jnp.zeros((LANES,), f32)

        for k in range(num_hop):
            if k > 0:
                flag_wait(k - 1)
                combine(pouts[k - 1], k - 1)
            pltpu.sync_copy(hbuf, h_sh.at[pl.ds(base, C)])
            pltpu.sync_copy(q0, agg_sh.at[pl.ds(base, C)])
            plsc.subcore_barrier()
            _edge_ring(src_ref, dst_ref, h_sh, agg_sh,
                       sbufs, dbufs, vbufs, sl, ss, (c * NS + s) * EPT, EPT)
            plsc.subcore_barrier()
            pltpu.sync_copy(agg_sh.at[pl.ds(base, C)], q0)
            pltpu.sync_copy(q0, pouts[k].at[pl.ds(c * n_pad + base, C)])
            plsc.subcore_barrier()

            @pl.when(s == 0)
            def _():
                fbuf[...] = jnp.full((FW,), k + 1, i32)
                pltpu.sync_copy(
                    fbuf,
                    flags_ref.at[pl.ds((c * n_flag_slots + k) * FW, FW)])

        flag_wait(num_hop - 1)
        combine(pouts[num_hop - 1], num_hop - 1)

        @pl.when(c == 0)
        def _():
            pltpu.sync_copy(hbuf, out_ref.at[pl.ds(base, C)])

    return pl.kernel(body, out_type=out_type, mesh=_MESH,
                     scratch_types=scratch, compiler_params=_CP)


def kernel(x, edge_index, W):
    n = x.shape[0]
    e = edge_index.shape[1]
    num_hop = W.shape[0]
    n_pad = -(-n // (NS * LANES)) * (NS * LANES)
    assert e % (NT * EB) == 0 and num_hop >= 1

    src = edge_index[0].astype(i32)
    dst = edge_index[1].astype(i32)
    wvec = jnp.broadcast_to(W[:, 0, 0].astype(f32)[:, None],
                            (num_hop, LANES)).reshape(-1)

    h = jnp.zeros((n_pad,), f32).at[:n].set(x[:, 0])

    flags = jnp.where(src[:2 * num_hop * FW] > jnp.int32(2**30),
                      jnp.int32(1), jnp.int32(0)) * jnp.int32(2 ** 20)

    out = _all_hops(n_pad, e, num_hop)(h, wvec, src, dst, flags)[0]

    return out[:n].reshape(n, 1)


# speedup vs baseline: 1.7136x; 1.2159x over previous
import jax
import jax.numpy as jnp
from jax import lax
from jax.experimental import pallas as pl
from jax.experimental.pallas import tpu as pltpu
from jax.experimental.pallas import tpu_sc as plsc

f32 = jnp.float32
i32 = jnp.int32

NC = 2
NS = 16
NT = NC * NS
LANES = 16
EB = 1600
FW = 16

_MESH = plsc.VectorSubcoreMesh(core_axis_name="c", subcore_axis_name="s")

_CP = pltpu.CompilerParams(needs_layout_passes=False)


def _ring_scratch():
    return [
        [pltpu.VMEM((EB,), i32)] * 4,
        [pltpu.VMEM((EB,), i32)] * 4,
        [pltpu.VMEM((EB,), f32)] * 2,
        [pltpu.SemaphoreType.DMA] * 4,
        [pltpu.SemaphoreType.DMA] * 2,
    ]


def _edge_ring(src_ref, dst_ref, h_tile, agg_sh, sbufs, dbufs, vbufs, sl, ss,
               ebase, ept):
    NBLK = ept // EB
    MAIN = (NBLK - 4) // 4 * 4
    assert NBLK >= 6

    def start_loads(blk, b4):
        off = ebase + blk * EB
        pltpu.async_copy(src_ref.at[pl.ds(off, EB)], sbufs[b4], sl[b4])
        pltpu.async_copy(dst_ref.at[pl.ds(off, EB)], dbufs[b4], sl[b4])

    def wait_loads(b4):
        pltpu.make_async_copy(
            src_ref.at[pl.ds(0, EB)], sbufs[b4], sl[b4]).wait()
        pltpu.make_async_copy(
            dst_ref.at[pl.ds(0, EB)], dbufs[b4], sl[b4]).wait()

    def gather(b4, b2):
        @pl.loop(0, EB, step=LANES)
        def _(i):
            idx = sbufs[b4][pl.ds(i, LANES)]
            vbufs[b2][pl.ds(i, LANES)] = plsc.load_gather(h_tile, [idx])

    def start_scatter(b4, b2):
        pltpu.async_copy(vbufs[b2], agg_sh.at[dbufs[b4]], ss[b2], add=True)

    def wait_scatter(b4, b2):
        pltpu.make_async_copy(vbufs[b2], agg_sh.at[dbufs[b4]], ss[b2]).wait()

    start_loads(0, 0)
    start_loads(1, 1)
    for blk in (0, 1):
        wait_loads(blk)
        gather(blk, blk)
        start_scatter(blk, blk)
        start_loads(blk + 2, blk + 2)

    @pl.loop(2, 2 + MAIN, step=4)
    def _(g):
        for j in range(4):
            blk_s = 2 + j
            b4, b2 = blk_s % 4, blk_s % 2
            wait_loads(b4)
            wait_scatter((blk_s - 2) % 4, b2)
            gather(b4, b2)
            start_scatter(b4, b2)
            start_loads(g + j + 2, (blk_s + 2) % 4)

    for blk in range(2 + MAIN, NBLK):
        b4, b2 = blk % 4, blk % 2
        wait_loads(b4)
        wait_scatter((blk - 2) % 4, b2)
        gather(b4, b2)
        start_scatter(b4, b2)
        if blk + 2 < NBLK:
            start_loads(blk + 2, (blk + 2) % 4)
    wait_scatter((NBLK - 2) % 4, (NBLK - 2) % 2)
    wait_scatter((NBLK - 1) % 4, (NBLK - 1) % 2)


def _all_hops(n_pad, e, num_hop):
    C = n_pad // NS
    EPT = e // NT
    n_flag_slots = num_hop

    out_type = tuple(
        [jax.ShapeDtypeStruct((n_pad,), f32)]
        + [jax.ShapeDtypeStruct((2 * n_pad,), f32)] * (n_flag_slots + 1))
    scratch = [
        pltpu.VMEM_SHARED((n_pad,), f32),
        pltpu.VMEM((n_pad,), f32),
        pltpu.VMEM((C,), f32),
        pltpu.VMEM((LANES,), f32),
        pltpu.VMEM((FW,), i32),
        pltpu.SemaphoreType.DMA,
    ] + _ring_scratch()

    def body(*refs):
        (h_ref, w_ref, src_ref, dst_ref, flags_ref,
         out_ref, *rest) = refs
        pouts = rest[:n_flag_slots]
        hrep_ref = rest[n_flag_slots]
        (agg_sh, h_tile, q0, wbuf, fbuf, sf,
         sbufs, dbufs, vbufs, sl, ss) = rest[n_flag_slots + 1:]
        c = lax.axis_index("c")
        s = lax.axis_index("s")
        base = s * C

        def combine(psrc, k):
            pltpu.sync_copy(w_ref.at[pl.ds(k * LANES, LANES)], wbuf)
            pltpu.sync_copy(psrc.at[pl.ds(base, C)], q0)

            @pl.loop(0, C, step=LANES)
            def _(i):
                hs = pl.ds(base + i, LANES)
                h_tile[hs] = h_tile[hs] + q0[pl.ds(i, LANES)]

            pltpu.sync_copy(psrc.at[pl.ds(n_pad + base, C)], q0)
            wv = wbuf[...]

            @pl.loop(0, C, step=LANES)
            def _(i):
                hs = pl.ds(base + i, LANES)
                sl_ = pl.ds(i, LANES)
                h_tile[hs] = (h_tile[hs] + q0[sl_]) * wv
                q0[sl_] = jnp.zeros((LANES,), f32)

        def flag_wait(k):
            off = ((1 - c) * n_flag_slots + k) * FW

            def cond(carry):
                it, done = carry
                return jnp.logical_and(done == 0, it < jnp.int32(200000))

            def poll(carry):
                it, _ = carry
                pltpu.async_copy(flags_ref.at[pl.ds(off, FW)], fbuf, sf
                                 ).wait()
                got = jnp.max(
                    jnp.where(fbuf[...] == k + 1, 1, 0).astype(i32))
                return (it + jnp.int32(1), got)

            lax.while_loop(cond, poll, (jnp.int32(0), jnp.int32(0)))

        pltpu.sync_copy(h_ref, h_tile)

        @pl.loop(0, C, step=LANES)
        def _(i):
            q0[pl.ds(i, LANES)] = jnp.zeros((LANES,), f32)

        for k in range(num_hop):
            if k > 0:
                flag_wait(k - 1)
                combine(pouts[k - 1], k - 1)
                pltpu.sync_copy(h_tile.at[pl.ds(base, C)],
                                hrep_ref.at[pl.ds(c * n_pad + base, C)])
            pltpu.sync_copy(q0, agg_sh.at[pl.ds(base, C)])
            plsc.subcore_barrier()
            if k > 0:
                pltpu.sync_copy(hrep_ref.at[pl.ds(c * n_pad, n_pad)], h_tile)
            _edge_ring(src_ref, dst_ref, h_tile, agg_sh,
                       sbufs, dbufs, vbufs, sl, ss, (c * NS + s) * EPT, EPT)
            plsc.subcore_barrier()
            pltpu.sync_copy(agg_sh.at[pl.ds(base, C)], q0)
            pltpu.sync_copy(q0, pouts[k].at[pl.ds(c * n_pad + base, C)])
            plsc.subcore_barrier()

            @pl.when(s == 0)
            def _():
                fbuf[...] = jnp.full((FW,), k + 1, i32)
                pltpu.sync_copy(
                    fbuf,
                    flags_ref.at[pl.ds((c * n_flag_slots + k) * FW, FW)])

        flag_wait(num_hop - 1)
        combine(pouts[num_hop - 1], num_hop - 1)

        @pl.when(c == 0)
        def _():
            pltpu.sync_copy(h_tile.at[pl.ds(base, C)],
                            out_ref.at[pl.ds(base, C)])

    return pl.kernel(body, out_type=out_type, mesh=_MESH,
                     scratch_types=scratch, compiler_params=_CP)


def kernel(x, edge_index, W):
    n = x.shape[0]
    e = edge_index.shape[1]
    num_hop = W.shape[0]
    n_pad = -(-n // (NS * LANES)) * (NS * LANES)
    assert e % (NT * EB) == 0 and num_hop >= 1

    src = edge_index[0].astype(i32)
    dst = edge_index[1].astype(i32)
    wvec = jnp.broadcast_to(W[:, 0, 0].astype(f32)[:, None],
                            (num_hop, LANES)).reshape(-1)

    h = jnp.zeros((n_pad,), f32).at[:n].set(x[:, 0])

    flags = jnp.where(src[:2 * num_hop * FW] > jnp.int32(2**30),
                      jnp.int32(1), jnp.int32(0)) * jnp.int32(2 ** 20)

    out = _all_hops(n_pad, e, num_hop)(h, wvec, src, dst, flags)[0]

    return out[:n].reshape(n, 1)


# speedup vs baseline: 2.1224x; 1.2386x over previous
import jax
import jax.numpy as jnp
from jax import lax
from jax.experimental import pallas as pl
from jax.experimental.pallas import tpu as pltpu
from jax.experimental.pallas import tpu_sc as plsc

f32 = jnp.float32
i32 = jnp.int32

NC = 2
NS = 16
NT = NC * NS
LANES = 16
EB = 1600
FW = 16

_MESH = plsc.VectorSubcoreMesh(core_axis_name="c", subcore_axis_name="s")

_CP = pltpu.CompilerParams(needs_layout_passes=False)


def _ring_scratch():
    return [
        [pltpu.VMEM((EB,), i32)] * 4,
        [pltpu.VMEM((EB,), i32)] * 4,
        [pltpu.VMEM((EB,), f32)] * 2,
        [pltpu.SemaphoreType.DMA] * 4,
        [pltpu.SemaphoreType.DMA] * 2,
    ]


def _edge_ring(src_ref, dst_ref, h_tile, agg_sh, sbufs, dbufs, vbufs, sl, ss,
               ebase, ept):
    NBLK = ept // EB
    MAIN = (NBLK - 4) // 4 * 4
    assert NBLK >= 6

    def start_loads(blk, b4):
        off = ebase + blk * EB
        pltpu.async_copy(src_ref.at[pl.ds(off, EB)], sbufs[b4], sl[b4])
        pltpu.async_copy(dst_ref.at[pl.ds(off, EB)], dbufs[b4], sl[b4])

    def wait_loads(b4):
        pltpu.make_async_copy(
            src_ref.at[pl.ds(0, EB)], sbufs[b4], sl[b4]).wait()
        pltpu.make_async_copy(
            dst_ref.at[pl.ds(0, EB)], dbufs[b4], sl[b4]).wait()

    def gather(b4, b2):
        @plsc.parallel_loop(0, EB, LANES, unroll=4)
        def _(i):
            idx = sbufs[b4][pl.ds(i, LANES)]
            vbufs[b2][pl.ds(i, LANES)] = plsc.load_gather(h_tile, [idx])

    def start_scatter(b4, b2):
        pltpu.async_copy(vbufs[b2], agg_sh.at[dbufs[b4]], ss[b2], add=True)

    def wait_scatter(b4, b2):
        pltpu.make_async_copy(vbufs[b2], agg_sh.at[dbufs[b4]], ss[b2]).wait()

    start_loads(0, 0)
    start_loads(1, 1)
    for blk in (0, 1):
        wait_loads(blk)
        gather(blk, blk)
        start_scatter(blk, blk)
        start_loads(blk + 2, blk + 2)

    @pl.loop(2, 2 + MAIN, step=4)
    def _(g):
        for j in range(4):
            blk_s = 2 + j
            b4, b2 = blk_s % 4, blk_s % 2
            wait_loads(b4)
            wait_scatter((blk_s - 2) % 4, b2)
            gather(b4, b2)
            start_scatter(b4, b2)
            start_loads(g + j + 2, (blk_s + 2) % 4)

    for blk in range(2 + MAIN, NBLK):
        b4, b2 = blk % 4, blk % 2
        wait_loads(b4)
        wait_scatter((blk - 2) % 4, b2)
        gather(b4, b2)
        start_scatter(b4, b2)
        if blk + 2 < NBLK:
            start_loads(blk + 2, (blk + 2) % 4)
    wait_scatter((NBLK - 2) % 4, (NBLK - 2) % 2)
    wait_scatter((NBLK - 1) % 4, (NBLK - 1) % 2)


def _all_hops(n_pad, e, num_hop):
    C = n_pad // NS
    EPT = e // NT
    n_flag_slots = num_hop

    out_type = tuple(
        [jax.ShapeDtypeStruct((n_pad,), f32)]
        + [jax.ShapeDtypeStruct((2 * n_pad,), f32)] * (n_flag_slots + 1))
    scratch = [
        pltpu.VMEM_SHARED((n_pad,), f32),
        pltpu.VMEM((n_pad,), f32),
        pltpu.VMEM((C,), f32),
        pltpu.VMEM((LANES,), f32),
        pltpu.VMEM((FW,), i32),
        pltpu.SemaphoreType.DMA,
    ] + _ring_scratch()

    def body(*refs):
        (h_ref, w_ref, src_ref, dst_ref, flags_ref,
         out_ref, *rest) = refs
        pouts = rest[:n_flag_slots]
        hrep_ref = rest[n_flag_slots]
        (agg_sh, h_tile, q0, wbuf, fbuf, sf,
         sbufs, dbufs, vbufs, sl, ss) = rest[n_flag_slots + 1:]
        c = lax.axis_index("c")
        s = lax.axis_index("s")
        base = s * C

        def combine(psrc, k):
            pltpu.sync_copy(w_ref.at[pl.ds(k * LANES, LANES)], wbuf)
            pltpu.sync_copy(psrc.at[pl.ds(base, C)], q0)

            @pl.loop(0, C, step=LANES)
            def _(i):
                hs = pl.ds(base + i, LANES)
                h_tile[hs] = h_tile[hs] + q0[pl.ds(i, LANES)]

            pltpu.sync_copy(psrc.at[pl.ds(n_pad + base, C)], q0)
            wv = wbuf[...]

            @pl.loop(0, C, step=LANES)
            def _(i):
                hs = pl.ds(base + i, LANES)
                sl_ = pl.ds(i, LANES)
                h_tile[hs] = (h_tile[hs] + q0[sl_]) * wv
                q0[sl_] = jnp.zeros((LANES,), f32)

        def flag_wait(k):
            off = ((1 - c) * n_flag_slots + k) * FW

            def cond(carry):
                it, done = carry
                return jnp.logical_and(done == 0, it < jnp.int32(200000))

            def poll(carry):
                it, _ = carry
                pltpu.async_copy(flags_ref.at[pl.ds(off, FW)], fbuf, sf
                                 ).wait()
                got = jnp.max(
                    jnp.where(fbuf[...] == k + 1, 1, 0).astype(i32))
                return (it + jnp.int32(1), got)

            lax.while_loop(cond, poll, (jnp.int32(0), jnp.int32(0)))

        pltpu.sync_copy(h_ref, h_tile)

        @pl.loop(0, C, step=LANES)
        def _(i):
            q0[pl.ds(i, LANES)] = jnp.zeros((LANES,), f32)

        for k in range(num_hop):
            if k > 0:
                flag_wait(k - 1)
                combine(pouts[k - 1], k - 1)
                pltpu.sync_copy(h_tile.at[pl.ds(base, C)],
                                hrep_ref.at[pl.ds(c * n_pad + base, C)])
            pltpu.sync_copy(q0, agg_sh.at[pl.ds(base, C)])
            plsc.subcore_barrier()
            if k > 0:
                pltpu.sync_copy(hrep_ref.at[pl.ds(c * n_pad, n_pad)], h_tile)
            _edge_ring(src_ref, dst_ref, h_tile, agg_sh,
                       sbufs, dbufs, vbufs, sl, ss, (c * NS + s) * EPT, EPT)
            plsc.subcore_barrier()
            pltpu.sync_copy(agg_sh.at[pl.ds(base, C)], q0)
            pltpu.sync_copy(q0, pouts[k].at[pl.ds(c * n_pad + base, C)])
            plsc.subcore_barrier()

            @pl.when(s == 0)
            def _():
                fbuf[...] = jnp.full((FW,), k + 1, i32)
                pltpu.sync_copy(
                    fbuf,
                    flags_ref.at[pl.ds((c * n_flag_slots + k) * FW, FW)])

        flag_wait(num_hop - 1)
        combine(pouts[num_hop - 1], num_hop - 1)

        @pl.when(c == 0)
        def _():
            pltpu.sync_copy(h_tile.at[pl.ds(base, C)],
                            out_ref.at[pl.ds(base, C)])

    return pl.kernel(body, out_type=out_type, mesh=_MESH,
                     scratch_types=scratch, compiler_params=_CP)


def kernel(x, edge_index, W):
    n = x.shape[0]
    e = edge_index.shape[1]
    num_hop = W.shape[0]
    n_pad = -(-n // (NS * LANES)) * (NS * LANES)
    assert e % (NT * EB) == 0 and num_hop >= 1

    src = edge_index[0].astype(i32)
    dst = edge_index[1].astype(i32)
    wvec = jnp.broadcast_to(W[:, 0, 0].astype(f32)[:, None],
                            (num_hop, LANES)).reshape(-1)

    h = jnp.zeros((n_pad,), f32).at[:n].set(x[:, 0])

    flags = jnp.where(src[:2 * num_hop * FW] > jnp.int32(2**30),
                      jnp.int32(1), jnp.int32(0)) * jnp.int32(2 ** 20)

    out = _all_hops(n_pad, e, num_hop)(h, wvec, src, dst, flags)[0]

    return out[:n].reshape(n, 1)
